# baseline (speedup 1.0000x reference)
import jax
import jax.numpy as jnp
from jax.experimental import pallas as pl
from jax.experimental.pallas import tpu as pltpu

WIDTH = 128
N_REL = 16
N_NODE = 512


def _body(x_ref, db_hbm, a1w_ref, a2w_ref, opw_ref, cw_ref,
          out_ref, dbv, sems):
    copies = [
        pltpu.make_async_copy(db_hbm.at[i], dbv.at[i], sems.at[i])
        for i in range(2 * N_REL)
    ]
    for c in copies:
        c.start()

    x = x_ref[...]
    w1 = a1w_ref[...]
    w1 = jnp.exp(w1 - jnp.max(w1, axis=0, keepdims=True))
    w1 = w1 / jnp.sum(w1, axis=0, keepdims=True)
    w2 = a2w_ref[...]
    w2 = jnp.exp(w2 - jnp.max(w2, axis=0, keepdims=True))
    w2 = w2 / jnp.sum(w2, axis=0, keepdims=True)
    arg1 = jax.lax.dot_general(
        w1, x, (((0,), (0,)), ((), ())), preferred_element_type=jnp.float32)
    arg2 = jax.lax.dot_general(
        w2, x, (((0,), (0,)), ((), ())), preferred_element_type=jnp.float32)
    cw = cw_ref[...]
    cw = jnp.exp(cw - jnp.max(cw, axis=1, keepdims=True))
    cwsm = cw / jnp.sum(cw, axis=1, keepdims=True)

    opw = opw_ref[...]
    opw = jnp.exp(opw - jnp.max(opw, axis=1, keepdims=True))
    opw = opw / jnp.sum(opw, axis=1, keepdims=True)

    HALF = N_NODE // 2
    acc_lo = jnp.zeros((WIDTH, HALF), jnp.float32)
    acc_hi = jnp.zeros((WIDTH, HALF), jnp.float32)
    for i in range(N_REL):
        xw1 = (arg2 * cwsm[:, i:i + 1]).astype(jnp.bfloat16)
        xw2 = (arg2 * cwsm[:, N_REL + i:N_REL + i + 1]).astype(jnp.bfloat16)
        for h in range(2):
            copies[2 * i + h].wait()
            d = dbv[2 * i + h].astype(jnp.bfloat16)
            fwd = jax.lax.dot_general(
                xw1[:, h * HALF:(h + 1) * HALF], d, (((1,), (0,)), ((), ())),
                preferred_element_type=jnp.float32)
            bwd = jax.lax.dot_general(
                xw2, d, (((1,), (1,)), ((), ())),
                preferred_element_type=jnp.float32)
            acc_lo = acc_lo + fwd[:, :HALF]
            acc_hi = acc_hi + fwd[:, HALF:]
            if h == 0:
                acc_lo = acc_lo + bwd
            else:
                acc_hi = acc_hi + bwd

    chain = 1.0 - jnp.exp(-jnp.concatenate([acc_lo, acc_hi], axis=1))
    a12 = arg1 * arg2
    out_ref[...] = (opw[:, 0:1] * arg2
                    + opw[:, 1:2] * a12
                    + opw[:, 2:3] * (arg1 + arg2 - a12)
                    + opw[:, 3:4] * chain
                    + opw[:, 4:5] * (1.0 - arg1))


def kernel(inputs, database, arg1_weights, arg2_weights, op_weights, chain_weights):
    return pl.pallas_call(
        _body,
        in_specs=[
            pl.BlockSpec(memory_space=pltpu.MemorySpace.VMEM),
            pl.BlockSpec(memory_space=pltpu.MemorySpace.HBM),
            pl.BlockSpec(memory_space=pltpu.MemorySpace.VMEM),
            pl.BlockSpec(memory_space=pltpu.MemorySpace.VMEM),
            pl.BlockSpec(memory_space=pltpu.MemorySpace.VMEM),
            pl.BlockSpec(memory_space=pltpu.MemorySpace.VMEM),
        ],
        out_specs=pl.BlockSpec(memory_space=pltpu.MemorySpace.VMEM),
        out_shape=jax.ShapeDtypeStruct((WIDTH, N_NODE), jnp.float32),
        scratch_shapes=[
            pltpu.VMEM((2 * N_REL, N_NODE // 2, N_NODE), jnp.float32),
            pltpu.SemaphoreType.DMA((2 * N_REL,)),
        ],
    )(inputs, database.reshape(2 * N_REL, N_NODE // 2, N_NODE), arg1_weights, arg2_weights, op_weights, chain_weights)


# speedup vs baseline: 1.1170x; 1.1170x over previous
import jax
import jax.numpy as jnp
from jax.experimental import pallas as pl
from jax.experimental.pallas import tpu as pltpu

WIDTH = 128
N_REL = 16
N_NODE = 512


def _body(x_ref, db_hbm, a1w_ref, a2w_ref, opw_ref, cw_ref,
          out_ref, dbv, sems):
    copies = [
        pltpu.make_async_copy(db_hbm.at[i], dbv.at[i], sems.at[i])
        for i in range(N_REL)
    ]
    for c in copies:
        c.start()

    x = x_ref[...]
    w1 = a1w_ref[...]
    w1 = jnp.exp(w1 - jnp.max(w1, axis=0, keepdims=True))
    w1 = w1 / jnp.sum(w1, axis=0, keepdims=True)
    w2 = a2w_ref[...]
    w2 = jnp.exp(w2 - jnp.max(w2, axis=0, keepdims=True))
    w2 = w2 / jnp.sum(w2, axis=0, keepdims=True)
    arg1 = jax.lax.dot_general(
        w1, x, (((0,), (0,)), ((), ())), preferred_element_type=jnp.float32)
    arg2 = jax.lax.dot_general(
        w2, x, (((0,), (0,)), ((), ())), preferred_element_type=jnp.float32)
    cw = cw_ref[...]
    cw = jnp.exp(cw - jnp.max(cw, axis=1, keepdims=True))
    cwsm = cw / jnp.sum(cw, axis=1, keepdims=True)

    opw = opw_ref[...]
    opw = jnp.exp(opw - jnp.max(opw, axis=1, keepdims=True))
    opw = opw / jnp.sum(opw, axis=1, keepdims=True)

    accf = jnp.zeros((WIDTH, N_NODE), jnp.float32)
    accb = jnp.zeros((WIDTH, N_NODE), jnp.float32)
    for i in range(N_REL):
        copies[i].wait()
        d = dbv[i].astype(jnp.bfloat16)
        xw1 = (arg2 * cwsm[:, i:i + 1]).astype(jnp.bfloat16)
        xw2 = (arg2 * cwsm[:, N_REL + i:N_REL + i + 1]).astype(jnp.bfloat16)
        fwd = jax.lax.dot_general(
            xw1, d, (((1,), (0,)), ((), ())),
            preferred_element_type=jnp.float32)
        bwd = jax.lax.dot_general(
            xw2, d, (((1,), (1,)), ((), ())),
            preferred_element_type=jnp.float32)
        accf = accf + fwd
        accb = accb + bwd

    chain = 1.0 - jnp.exp(-(accf + accb))
    a12 = arg1 * arg2
    out_ref[...] = (opw[:, 0:1] * arg2
                    + opw[:, 1:2] * a12
                    + opw[:, 2:3] * (arg1 + arg2 - a12)
                    + opw[:, 3:4] * chain
                    + opw[:, 4:5] * (1.0 - arg1))


def kernel(inputs, database, arg1_weights, arg2_weights, op_weights, chain_weights):
    return pl.pallas_call(
        _body,
        in_specs=[
            pl.BlockSpec(memory_space=pltpu.MemorySpace.VMEM),
            pl.BlockSpec(memory_space=pltpu.MemorySpace.HBM),
            pl.BlockSpec(memory_space=pltpu.MemorySpace.VMEM),
            pl.BlockSpec(memory_space=pltpu.MemorySpace.VMEM),
            pl.BlockSpec(memory_space=pltpu.MemorySpace.VMEM),
            pl.BlockSpec(memory_space=pltpu.MemorySpace.VMEM),
        ],
        out_specs=pl.BlockSpec(memory_space=pltpu.MemorySpace.VMEM),
        out_shape=jax.ShapeDtypeStruct((WIDTH, N_NODE), jnp.float32),
        scratch_shapes=[
            pltpu.VMEM((N_REL, N_NODE, N_NODE), jnp.float32),
            pltpu.SemaphoreType.DMA((N_REL,)),
        ],
    )(inputs, database, arg1_weights, arg2_weights, op_weights, chain_weights)


# speedup vs baseline: 1.1328x; 1.0141x over previous
import jax
import jax.numpy as jnp
from jax.experimental import pallas as pl
from jax.experimental.pallas import tpu as pltpu

WIDTH = 128
N_REL = 16
N_NODE = 512


def _body(x_ref, db_hbm, a1w_ref, a2w_ref, opw_ref, cw_ref,
          out_ref, dbv, sems):
    copies = [
        pltpu.make_async_copy(db_hbm.at[i], dbv.at[i], sems.at[i])
        for i in range(N_REL)
    ]
    for c in copies:
        c.start()

    x = x_ref[...]
    w1 = a1w_ref[...]
    w1 = jnp.exp(w1 - jnp.max(w1, axis=0, keepdims=True))
    w1 = w1 / jnp.sum(w1, axis=0, keepdims=True)
    w2 = a2w_ref[...]
    w2 = jnp.exp(w2 - jnp.max(w2, axis=0, keepdims=True))
    w2 = w2 / jnp.sum(w2, axis=0, keepdims=True)
    arg1 = jax.lax.dot_general(
        w1, x, (((0,), (0,)), ((), ())), preferred_element_type=jnp.float32)
    arg2 = jax.lax.dot_general(
        w2, x, (((0,), (0,)), ((), ())), preferred_element_type=jnp.float32)
    cw = cw_ref[...]
    cw = jnp.exp(cw - jnp.max(cw, axis=1, keepdims=True))
    cwsm = cw / jnp.sum(cw, axis=1, keepdims=True)

    opw = opw_ref[...]
    opw = jnp.exp(opw - jnp.max(opw, axis=1, keepdims=True))
    opw = opw / jnp.sum(opw, axis=1, keepdims=True)

    acc = jnp.zeros((WIDTH, N_NODE), jnp.float32)
    for i in range(N_REL):
        copies[i].wait()
        d = dbv[i].astype(jnp.bfloat16)
        xw1 = (arg2 * cwsm[:, i:i + 1]).astype(jnp.bfloat16)
        xw2 = (arg2 * cwsm[:, N_REL + i:N_REL + i + 1]).astype(jnp.bfloat16)
        fwd = jax.lax.dot_general(
            xw1, d, (((1,), (0,)), ((), ())),
            preferred_element_type=jnp.float32)
        bwd = jax.lax.dot_general(
            xw2, d, (((1,), (1,)), ((), ())),
            preferred_element_type=jnp.float32)
        acc = acc + fwd + bwd

    chain = 1.0 - jnp.exp(-acc)
    a12 = arg1 * arg2
    out_ref[...] = (opw[:, 0:1] * arg2
                    + opw[:, 1:2] * a12
                    + opw[:, 2:3] * (arg1 + arg2 - a12)
                    + opw[:, 3:4] * chain
                    + opw[:, 4:5] * (1.0 - arg1))


def kernel(inputs, database, arg1_weights, arg2_weights, op_weights, chain_weights):
    return pl.pallas_call(
        _body,
        in_specs=[
            pl.BlockSpec(memory_space=pltpu.MemorySpace.VMEM),
            pl.BlockSpec(memory_space=pltpu.MemorySpace.HBM),
            pl.BlockSpec(memory_space=pltpu.MemorySpace.VMEM),
            pl.BlockSpec(memory_space=pltpu.MemorySpace.VMEM),
            pl.BlockSpec(memory_space=pltpu.MemorySpace.VMEM),
            pl.BlockSpec(memory_space=pltpu.MemorySpace.VMEM),
        ],
        out_specs=pl.BlockSpec(memory_space=pltpu.MemorySpace.VMEM),
        out_shape=jax.ShapeDtypeStruct((WIDTH, N_NODE), jnp.float32),
        scratch_shapes=[
            pltpu.VMEM((N_REL, N_NODE, N_NODE), jnp.float32),
            pltpu.SemaphoreType.DMA((N_REL,)),
        ],
    )(inputs, database, arg1_weights, arg2_weights, op_weights, chain_weights)
